# baseline (speedup 1.0000x reference)
import jax
import jax.numpy as jnp
from jax import lax
from jax.experimental import pallas as pl
from jax.experimental.pallas import tpu as pltpu
from jax.experimental.pallas import tpu_sc as plsc

VOCAB = 100000
MAXLEN = 200
DIM = 64
BATCH = 4096

NUM_CORES = 2
NUM_SUBCORES = 16
NW = NUM_CORES * NUM_SUBCORES

ROWS = BATCH * MAXLEN
ROWS_PER_W = ROWS // NW
CHUNK = 256
GATHER = 256
N_GATHER = CHUNK // GATHER
N_CHUNK = ROWS_PER_W // CHUNK
NBUF = 2
N_GROUP = N_CHUNK // NBUF
DIST = 1
LANES = 16
DSUB = DIM // LANES
RING = 512
IDX_ROWS = ROWS_PER_W // GATHER
PDIM = 2 * DIM


def _sc_body(x_hbm, tok_hbm, pos_hbm, out_hbm,
             idx_v, rows_v, stage_v, ring_v,
             gsem0, gsem1, osem0, osem1):
    gsems = [gsem0, gsem1]
    osems = [osem0, osem1]

    cid = lax.axis_index("c")
    sid = lax.axis_index("s")
    wid = sid * NUM_CORES + cid
    base = wid * ROWS_PER_W

    xrow = pl.multiple_of(wid * IDX_ROWS, 8)
    pltpu.sync_copy(x_hbm.at[pl.ds(xrow, IDX_ROWS)], idx_v)

    pltpu.sync_copy(pos_hbm, ring_v.at[pl.ds(0, MAXLEN)])
    pltpu.sync_copy(pos_hbm, ring_v.at[pl.ds(MAXLEN, MAXLEN)])
    pltpu.sync_copy(pos_hbm.at[pl.ds(0, RING - 2 * MAXLEN)],
                    ring_v.at[pl.ds(2 * MAXLEN, RING - 2 * MAXLEN)])

    def fire_chunk(ci, b):
        for t in range(N_GATHER):
            pltpu.async_copy(
                tok_hbm.at[idx_v.at[ci * N_GATHER + t]],
                rows_v.at[b, pl.ds(t * GATHER, GATHER)],
                gsems[b],
            )

    def wait_gathers(b):
        pltpu.make_async_copy(
            tok_hbm.at[pl.ds(0, CHUNK)], rows_v.at[b], gsems[b]
        ).wait()

    def fire_write(ci, b):
        off = pl.multiple_of((base + ci * CHUNK) // 2, CHUNK // 2)
        pltpu.async_copy(stage_v.at[b],
                         out_hbm.at[pl.ds(off, CHUNK // 2)], osems[b])

    def wait_write(b):
        pltpu.make_async_copy(
            stage_v.at[b], out_hbm.at[pl.ds(0, CHUNK // 2)], osems[b]
        ).wait()

    def add_pos(ci, b):
        m0 = lax.rem(ci * CHUNK, MAXLEN)

        @plsc.parallel_loop(0, CHUNK // 2, unroll=4)
        def _(p):
            r = 2 * p
            for h in range(2):
                for jj in range(DSUB):
                    sl = pl.ds(jj * LANES, LANES)
                    so = pl.ds(h * DIM + jj * LANES, LANES)
                    stage_v[b, p, so] = rows_v[b, r + h, sl] + ring_v[m0 + r + h, sl]

    for c in range(DIST):
        fire_chunk(c, c)

    def group_body(g, carry):
        for b in range(NBUF):
            i = g * NBUF + b
            wait_gathers(b)

            pb = (b + DIST) % NBUF
            if b + DIST < NBUF:
                @pl.when(g >= 1)
                def _():
                    wait_write(pb)
                fire_chunk(g * NBUF + b + DIST, pb)
            else:
                @pl.when(g + 1 < N_GROUP)
                def _():
                    wait_write(pb)
                    fire_chunk((g + 1) * NBUF + (b + DIST - NBUF), pb)

            add_pos(i, b)
            fire_write(i, b)
        return carry

    lax.fori_loop(0, N_GROUP, group_body, 0)

    for b in range(NBUF):
        wait_write(b)


@jax.jit
def kernel(x, token_table, pos_table):
    x_flat = x.reshape(ROWS // GATHER, GATHER).astype(jnp.int32)
    mesh = plsc.VectorSubcoreMesh(
        core_axis_name="c", subcore_axis_name="s",
        num_cores=NUM_CORES, num_subcores=NUM_SUBCORES,
    )
    out = pl.kernel(
        _sc_body,
        out_type=jax.ShapeDtypeStruct((ROWS // 2, PDIM), jnp.float32),
        mesh=mesh,
        scratch_types=[
            pltpu.VMEM((IDX_ROWS, GATHER), jnp.int32),
            pltpu.VMEM((NBUF, CHUNK, DIM), jnp.float32),
            pltpu.VMEM((NBUF, CHUNK // 2, PDIM), jnp.float32),
            pltpu.VMEM((RING, DIM), jnp.float32),
        ] + [pltpu.SemaphoreType.DMA] * (2 * NBUF),
        compiler_params=pltpu.CompilerParams(use_tc_tiling_on_sc=False),
    )(x_flat, token_table, pos_table)
    return out.reshape(BATCH, MAXLEN, DIM)


# speedup vs baseline: 1.0030x; 1.0030x over previous
import jax
import jax.numpy as jnp
from jax import lax
from jax.experimental import pallas as pl
from jax.experimental.pallas import tpu as pltpu
from jax.experimental.pallas import tpu_sc as plsc

VOCAB = 100000
MAXLEN = 200
DIM = 64
BATCH = 4096

NUM_CORES = 2
NUM_SUBCORES = 16
NW = NUM_CORES * NUM_SUBCORES

ROWS = BATCH * MAXLEN
ROWS_PER_W = ROWS // NW
CHUNK = 256
GATHER = 128
N_GATHER = CHUNK // GATHER
N_CHUNK = ROWS_PER_W // CHUNK
NBUF = 2
N_GROUP = N_CHUNK // NBUF
DIST = 1
LANES = 16
DSUB = DIM // LANES
RING = 512
IDX_ROWS = ROWS_PER_W // GATHER
PDIM = 2 * DIM


def _sc_body(x_hbm, tok_hbm, pos_hbm, out_hbm,
             idx_v, rows_v, stage_v, ring_v,
             gsem0, gsem1, osem0, osem1):
    gsems = [gsem0, gsem1]
    osems = [osem0, osem1]

    cid = lax.axis_index("c")
    sid = lax.axis_index("s")
    wid = sid * NUM_CORES + cid
    base = wid * ROWS_PER_W

    xrow = pl.multiple_of(wid * IDX_ROWS, 8)
    pltpu.sync_copy(x_hbm.at[pl.ds(xrow, IDX_ROWS)], idx_v)

    pltpu.sync_copy(pos_hbm, ring_v.at[pl.ds(0, MAXLEN)])
    pltpu.sync_copy(pos_hbm, ring_v.at[pl.ds(MAXLEN, MAXLEN)])
    pltpu.sync_copy(pos_hbm.at[pl.ds(0, RING - 2 * MAXLEN)],
                    ring_v.at[pl.ds(2 * MAXLEN, RING - 2 * MAXLEN)])

    def fire_chunk(ci, b):
        for t in range(N_GATHER):
            pltpu.async_copy(
                tok_hbm.at[idx_v.at[ci * N_GATHER + t]],
                rows_v.at[b, pl.ds(t * GATHER, GATHER)],
                gsems[b],
            )

    def wait_gathers(b):
        pltpu.make_async_copy(
            tok_hbm.at[pl.ds(0, CHUNK)], rows_v.at[b], gsems[b]
        ).wait()

    def fire_write(ci, b):
        off = pl.multiple_of((base + ci * CHUNK) // 2, CHUNK // 2)
        pltpu.async_copy(stage_v.at[b],
                         out_hbm.at[pl.ds(off, CHUNK // 2)], osems[b])

    def wait_write(b):
        pltpu.make_async_copy(
            stage_v.at[b], out_hbm.at[pl.ds(0, CHUNK // 2)], osems[b]
        ).wait()

    def add_pos(ci, b):
        m0 = lax.rem(ci * CHUNK, MAXLEN)

        @plsc.parallel_loop(0, CHUNK // 2, unroll=4)
        def _(p):
            r = 2 * p
            for h in range(2):
                for jj in range(DSUB):
                    sl = pl.ds(jj * LANES, LANES)
                    so = pl.ds(h * DIM + jj * LANES, LANES)
                    stage_v[b, p, so] = rows_v[b, r + h, sl] + ring_v[m0 + r + h, sl]

    for c in range(DIST):
        fire_chunk(c, c)

    def group_body(g, carry):
        for b in range(NBUF):
            i = g * NBUF + b
            wait_gathers(b)

            pb = (b + DIST) % NBUF
            if b + DIST < NBUF:
                @pl.when(g >= 1)
                def _():
                    wait_write(pb)
                fire_chunk(g * NBUF + b + DIST, pb)
            else:
                @pl.when(g + 1 < N_GROUP)
                def _():
                    wait_write(pb)
                    fire_chunk((g + 1) * NBUF + (b + DIST - NBUF), pb)

            add_pos(i, b)
            fire_write(i, b)
        return carry

    lax.fori_loop(0, N_GROUP, group_body, 0)

    for b in range(NBUF):
        wait_write(b)


@jax.jit
def kernel(x, token_table, pos_table):
    x_flat = x.reshape(ROWS // GATHER, GATHER).astype(jnp.int32)
    mesh = plsc.VectorSubcoreMesh(
        core_axis_name="c", subcore_axis_name="s",
        num_cores=NUM_CORES, num_subcores=NUM_SUBCORES,
    )
    out = pl.kernel(
        _sc_body,
        out_type=jax.ShapeDtypeStruct((ROWS // 2, PDIM), jnp.float32),
        mesh=mesh,
        scratch_types=[
            pltpu.VMEM((IDX_ROWS, GATHER), jnp.int32),
            pltpu.VMEM((NBUF, CHUNK, DIM), jnp.float32),
            pltpu.VMEM((NBUF, CHUNK // 2, PDIM), jnp.float32),
            pltpu.VMEM((RING, DIM), jnp.float32),
        ] + [pltpu.SemaphoreType.DMA] * (2 * NBUF),
        compiler_params=pltpu.CompilerParams(use_tc_tiling_on_sc=False),
    )(x_flat, token_table, pos_table)
    return out.reshape(BATCH, MAXLEN, DIM)
